# baseline (speedup 1.0000x reference)
import jax
import jax.numpy as jnp
from jax.experimental import pallas as pl
from jax.experimental.pallas import tpu as pltpu

B = 4
C = 96
H = 224
W = 224
TOPK = 10
THRESHOLD = 0.5
MARGIN = 0.5
HB = 32
NSTEP = H // HB
NEG_INF = float("-inf")
PAD_BASE = 1.0e6
INT_BIG = 2**31 - 1


def _pool3(x):
    ninf_col = jnp.full((H, 1), NEG_INF, jnp.float32)
    ninf_row = jnp.full((1, W), NEG_INF, jnp.float32)
    m = jnp.maximum(x, jnp.concatenate([x[:, 1:], ninf_col], axis=1))
    m = jnp.maximum(m, jnp.concatenate([ninf_col, x[:, :-1]], axis=1))
    p = jnp.maximum(m, jnp.concatenate([m[1:, :], ninf_row], axis=0))
    p = jnp.maximum(p, jnp.concatenate([ninf_row, m[:-1, :]], axis=0))
    return p


def _make_vals(x, vals_ref, m):
    pooled = _pool3(x)
    mask = (x == pooled) & (x > THRESHOLD)
    flat = (
        jax.lax.broadcasted_iota(jnp.int32, (H, W), 0) * W
        + jax.lax.broadcasted_iota(jnp.int32, (H, W), 1)
    )
    vals_ref[m] = jnp.where(mask, x, -(PAD_BASE + flat.astype(jnp.float32)))


HALF_W = (128, 96)


def _half_topk(vals_ref, svals_ref, sidx_ref, m, half):
    width = HALF_W[half]
    base = half * 128
    riota = jax.lax.broadcasted_iota(jnp.int32, (8, width), 0)
    ciota = jax.lax.broadcasted_iota(jnp.int32, (8, width), 1)
    rel = riota * W + ciota + base
    M = [jnp.full((8, width), NEG_INF, jnp.float32) for _ in range(TOPK)]
    I = [jnp.zeros((8, width), jnp.int32) for _ in range(TOPK)]
    for i in range(H // 8):
        v = vals_ref[m, pl.ds(8 * i, 8), pl.ds(base, width)]
        iv = rel + (8 * i * W)
        for j in range(TOPK):
            cmp = v > M[j]
            nm = jnp.where(cmp, v, M[j])
            ni = jnp.where(cmp, iv, I[j])
            v = jnp.where(cmp, M[j], v)
            iv = jnp.where(cmp, I[j], iv)
            M[j] = nm
            I[j] = ni
    for k in range(TOPK):
        mx = jnp.max(M[0])
        cand = M[0] == mx
        ix = jnp.min(jnp.where(cand, I[0], INT_BIG))
        svals_ref[m, half, k] = mx
        sidx_ref[m, half, k] = ix
        pos = cand & (I[0] == ix)
        for j in range(TOPK - 1):
            M[j] = jnp.where(pos, M[j + 1], M[j])
            I[j] = jnp.where(pos, I[j + 1], I[j])
        M[TOPK - 1] = jnp.where(pos, NEG_INF, M[TOPK - 1])


def _merge_halves(svals_ref, sidx_ref, idx_ref):
    for m in range(2):
        ai = jnp.int32(0)
        bi = jnp.int32(0)
        for k in range(TOPK):
            va = svals_ref[m, 0, ai]
            vb = svals_ref[m, 1, bi]
            ia = sidx_ref[m, 0, ai]
            ib = sidx_ref[m, 1, bi]
            take_a = (va > vb) | ((va == vb) & (ia < ib))
            idx_ref[m, k] = jnp.where(take_a, ia, ib)
            ai = ai + take_a.astype(jnp.int32)
            bi = bi + (1 - take_a.astype(jnp.int32))


def _issue_gathers(det_hbm, loc_hbm, b, idx_ref, rows_ref, sem):
    copies = []
    for m, hbm in ((0, det_hbm), (1, loc_hbm)):
        for k in range(TOPK):
            h = idx_ref[m, k] // W
            cp = pltpu.make_async_copy(
                hbm.at[b, :, h, :], rows_ref.at[m, k], sem)
            cp.start()
            copies.append(cp)
    for cp in copies:
        cp.wait()


def _select_feats(idx_ref, rows_ref, feats_ref):
    lane = jax.lax.broadcasted_iota(jnp.int32, (C, W), 1)
    for m in range(2):
        feats = []
        for k in range(TOPK):
            w = idx_ref[m, k] % W
            feats.append(
                jnp.sum(jnp.where(lane == w, rows_ref[m, k], 0.0), axis=1))
        feats_ref[m, 0:TOPK, :] = jnp.stack(feats, axis=0)


def _cosine_loss(feats_ref):
    df = feats_ref[0, 0:TOPK, :]
    lf = feats_ref[1, 0:TOPK, :]
    dn = jnp.maximum(jnp.sqrt(jnp.sum(df * df, axis=1)), 1e-8)
    ln = jnp.maximum(jnp.sqrt(jnp.sum(lf * lf, axis=1)), 1e-8)
    dots = jnp.sum(df[:, None, :] * lf[None, :, :], axis=2)
    sim = dots / (dn[:, None] * ln[None, :])
    return jnp.mean(jnp.maximum(sim - MARGIN, 0.0))


def _fused_body(loc_blk, det_blk, loc_hbm, det_hbm, out_ref,
                int_ref, vals_ref, svals_ref, sidx_ref, idx_ref, rows_ref,
                feats_ref, loss_ref, sem):
    b = pl.program_id(0)
    s = pl.program_id(1)

    @pl.when((b == 0) & (s == 0))
    def _init():
        loss_ref[0] = jnp.float32(0.0)

    @pl.when(b < B)
    def _intensity():
        lb = loc_blk[0]
        db = det_blk[0]
        row = pl.multiple_of(s * HB, HB)
        buf = jax.lax.rem(b, 2)
        int_ref[buf, 0, pl.ds(row, HB), :] = jnp.sqrt(jnp.sum(db * db, 0))
        int_ref[buf, 1, pl.ds(row, HB), :] = jnp.sqrt(jnp.sum(lb * lb, 0))

    @pl.when(b >= 1)
    def _phase2():
        prev = b - 1
        buf = jax.lax.rem(prev, 2)

        @pl.when(s == 0)
        def _():
            _make_vals(int_ref[buf, 0], vals_ref, 0)
            _make_vals(int_ref[buf, 1], vals_ref, 1)

        @pl.when(s == 1)
        def _():
            _half_topk(vals_ref, svals_ref, sidx_ref, 0, 0)

        @pl.when(s == 2)
        def _():
            _half_topk(vals_ref, svals_ref, sidx_ref, 0, 1)

        @pl.when(s == 3)
        def _():
            _half_topk(vals_ref, svals_ref, sidx_ref, 1, 0)

        @pl.when(s == 4)
        def _():
            _half_topk(vals_ref, svals_ref, sidx_ref, 1, 1)

        @pl.when(s == 5)
        def _():
            _merge_halves(svals_ref, sidx_ref, idx_ref)
            _issue_gathers(det_hbm, loc_hbm, prev, idx_ref, rows_ref, sem)

        @pl.when(s == 6)
        def _():
            _select_feats(idx_ref, rows_ref, feats_ref)
            loss_ref[0] = loss_ref[0] + _cosine_loss(feats_ref)

        @pl.when((s == 6) & (b == B))
        def _():
            out_ref[0, 0] = loss_ref[0] / B


def kernel(loc_features, det_features, gt_boxes):
    def in_map(b, s):
        bb = jnp.minimum(b, B - 1)
        ss = jnp.where(b >= B, NSTEP - 1, s)
        return (bb, 0, ss, 0)

    out = pl.pallas_call(
        _fused_body,
        grid=(B + 1, NSTEP),
        in_specs=[
            pl.BlockSpec((1, C, HB, W), in_map),
            pl.BlockSpec((1, C, HB, W), in_map),
            pl.BlockSpec(memory_space=pltpu.MemorySpace.HBM),
            pl.BlockSpec(memory_space=pltpu.MemorySpace.HBM),
        ],
        out_specs=pl.BlockSpec(
            (1, 1), lambda b, s: (0, 0),
            memory_space=pltpu.MemorySpace.SMEM),
        out_shape=jax.ShapeDtypeStruct((1, 1), jnp.float32),
        scratch_shapes=[
            pltpu.VMEM((2, 2, H, W), jnp.float32),
            pltpu.VMEM((2, H, W), jnp.float32),
            pltpu.SMEM((2, 2, TOPK), jnp.float32),
            pltpu.SMEM((2, 2, TOPK), jnp.int32),
            pltpu.SMEM((2, TOPK), jnp.int32),
            pltpu.VMEM((2, TOPK, C, W), jnp.float32),
            pltpu.VMEM((2, 16, C), jnp.float32),
            pltpu.SMEM((1,), jnp.float32),
            pltpu.SemaphoreType.DMA,
        ],
        compiler_params=pltpu.CompilerParams(
            dimension_semantics=("arbitrary", "arbitrary"),
        ),
    )(loc_features, det_features, loc_features, det_features)
    return out[0, 0]


# speedup vs baseline: 1.1814x; 1.1814x over previous
import jax
import jax.numpy as jnp
from jax.experimental import pallas as pl
from jax.experimental.pallas import tpu as pltpu

B = 4
C = 96
H = 224
W = 224
TOPK = 10
THRESHOLD = 0.5
MARGIN = 0.5
HB = 32
NSTEP = H // HB
NEG_INF = float("-inf")
PAD_BASE = 1.0e6
INT_BIG = 2**31 - 1


def _pool3(x):
    ninf_col = jnp.full((H, 1), NEG_INF, jnp.float32)
    ninf_row = jnp.full((1, W), NEG_INF, jnp.float32)
    m = jnp.maximum(x, jnp.concatenate([x[:, 1:], ninf_col], axis=1))
    m = jnp.maximum(m, jnp.concatenate([ninf_col, x[:, :-1]], axis=1))
    p = jnp.maximum(m, jnp.concatenate([m[1:, :], ninf_row], axis=0))
    p = jnp.maximum(p, jnp.concatenate([ninf_row, m[:-1, :]], axis=0))
    return p


def _make_vals(x, vals_ref, m):
    pooled = _pool3(x)
    mask = (x == pooled) & (x > THRESHOLD)
    flat = (
        jax.lax.broadcasted_iota(jnp.int32, (H, W), 0) * W
        + jax.lax.broadcasted_iota(jnp.int32, (H, W), 1)
    )
    vals_ref[m] = jnp.where(mask, x, -(PAD_BASE + flat.astype(jnp.float32)))


def _topk_map(vals_ref, idx_ref, m):
    riota = jax.lax.broadcasted_iota(jnp.int32, (8, W), 0)
    ciota = jax.lax.broadcasted_iota(jnp.int32, (8, W), 1)
    rel = riota * W + ciota
    M = [jnp.full((8, W), NEG_INF, jnp.float32) for _ in range(TOPK)]
    I = [jnp.zeros((8, W), jnp.int32) for _ in range(TOPK)]
    for i in range(H // 8):
        v = vals_ref[m, pl.ds(8 * i, 8), :]
        iv = rel + (8 * i * W)
        for j in range(TOPK):
            cmp = v > M[j]
            nm = jnp.where(cmp, v, M[j])
            ni = jnp.where(cmp, iv, I[j])
            v = jnp.where(cmp, M[j], v)
            iv = jnp.where(cmp, I[j], iv)
            M[j] = nm
            I[j] = ni
    for k in range(TOPK):
        mx = jnp.max(M[0])
        cand = M[0] == mx
        ix = jnp.min(jnp.where(cand, I[0], INT_BIG))
        idx_ref[m, k] = ix
        pos = cand & (I[0] == ix)
        for j in range(TOPK - 1):
            M[j] = jnp.where(pos, M[j + 1], M[j])
            I[j] = jnp.where(pos, I[j + 1], I[j])
        M[TOPK - 1] = jnp.where(pos, NEG_INF, M[TOPK - 1])


def _issue_gathers(det_hbm, loc_hbm, b, idx_ref, rows_ref, sem):
    copies = []
    for m, hbm in ((0, det_hbm), (1, loc_hbm)):
        for k in range(TOPK):
            h = idx_ref[m, k] // W
            cp = pltpu.make_async_copy(
                hbm.at[b, :, h, :], rows_ref.at[m, k], sem)
            cp.start()
            copies.append(cp)
    for cp in copies:
        cp.wait()


def _select_feats(idx_ref, rows_ref, feats_ref):
    lane = jax.lax.broadcasted_iota(jnp.int32, (C, W), 1)
    for m in range(2):
        feats = []
        for k in range(TOPK):
            w = idx_ref[m, k] % W
            feats.append(
                jnp.sum(jnp.where(lane == w, rows_ref[m, k], 0.0), axis=1))
        feats_ref[m, 0:TOPK, :] = jnp.stack(feats, axis=0)


def _cosine_loss(feats_ref):
    df = feats_ref[0, 0:TOPK, :]
    lf = feats_ref[1, 0:TOPK, :]
    dn = jnp.maximum(jnp.sqrt(jnp.sum(df * df, axis=1)), 1e-8)
    ln = jnp.maximum(jnp.sqrt(jnp.sum(lf * lf, axis=1)), 1e-8)
    dots = jnp.sum(df[:, None, :] * lf[None, :, :], axis=2)
    sim = dots / (dn[:, None] * ln[None, :])
    return jnp.mean(jnp.maximum(sim - MARGIN, 0.0))


def _fused_body(loc_blk, det_blk, loc_hbm, det_hbm, out_ref,
                int_ref, vals_ref, idx_ref, rows_ref,
                feats_ref, loss_ref, sem):
    b = pl.program_id(0)
    s = pl.program_id(1)

    @pl.when((b == 0) & (s == 0))
    def _init():
        loss_ref[0] = jnp.float32(0.0)

    @pl.when(b < B)
    def _intensity():
        lb = loc_blk[0]
        db = det_blk[0]
        row = pl.multiple_of(s * HB, HB)
        buf = jax.lax.rem(b, 2)
        int_ref[buf, 0, pl.ds(row, HB), :] = jnp.sqrt(jnp.sum(db * db, 0))
        int_ref[buf, 1, pl.ds(row, HB), :] = jnp.sqrt(jnp.sum(lb * lb, 0))

    @pl.when(b >= 1)
    def _phase2():
        prev = b - 1
        buf = jax.lax.rem(prev, 2)

        @pl.when(s == 0)
        def _():
            _make_vals(int_ref[buf, 0], vals_ref, 0)
            _make_vals(int_ref[buf, 1], vals_ref, 1)

        @pl.when(s == 1)
        def _():
            _topk_map(vals_ref, idx_ref, 0)

        @pl.when(s == 2)
        def _():
            _topk_map(vals_ref, idx_ref, 1)

        @pl.when(s == 3)
        def _():
            _issue_gathers(det_hbm, loc_hbm, prev, idx_ref, rows_ref, sem)

        @pl.when(s == 4)
        def _():
            _select_feats(idx_ref, rows_ref, feats_ref)

        @pl.when(s == 5)
        def _():
            loss_ref[0] = loss_ref[0] + _cosine_loss(feats_ref)

        @pl.when((s == 6) & (b == B))
        def _():
            out_ref[0, 0] = loss_ref[0] / B


def kernel(loc_features, det_features, gt_boxes):
    def in_map(b, s):
        bb = jnp.minimum(b, B - 1)
        ss = jnp.where(b >= B, NSTEP - 1, s)
        return (bb, 0, ss, 0)

    out = pl.pallas_call(
        _fused_body,
        grid=(B + 1, NSTEP),
        in_specs=[
            pl.BlockSpec((1, C, HB, W), in_map),
            pl.BlockSpec((1, C, HB, W), in_map),
            pl.BlockSpec(memory_space=pltpu.MemorySpace.HBM),
            pl.BlockSpec(memory_space=pltpu.MemorySpace.HBM),
        ],
        out_specs=pl.BlockSpec(
            (1, 1), lambda b, s: (0, 0),
            memory_space=pltpu.MemorySpace.SMEM),
        out_shape=jax.ShapeDtypeStruct((1, 1), jnp.float32),
        scratch_shapes=[
            pltpu.VMEM((2, 2, H, W), jnp.float32),
            pltpu.VMEM((2, H, W), jnp.float32),
            pltpu.SMEM((2, TOPK), jnp.int32),
            pltpu.VMEM((2, TOPK, C, W), jnp.float32),
            pltpu.VMEM((2, 16, C), jnp.float32),
            pltpu.SMEM((1,), jnp.float32),
            pltpu.SemaphoreType.DMA,
        ],
        compiler_params=pltpu.CompilerParams(
            dimension_semantics=("arbitrary", "arbitrary"),
        ),
    )(loc_features, det_features, loc_features, det_features)
    return out[0, 0]
